# baseline (speedup 1.0000x reference)
import jax
import jax.numpy as jnp
from jax.experimental import pallas as pl

N_NODES = 100000
N_SPECIES = 64
SUB = 8
LCH = 12800


def _onehot_t_body(idx_ref, out_ref):
    i = pl.program_id(0)
    row = idx_ref[...]
    sp = jax.lax.broadcasted_iota(jnp.int32, (SUB, LCH), 0) + SUB * i
    out_ref[...] = (row == sp).astype(jnp.float32)


def kernel(atom_types):
    idx_t = atom_types.T
    out_t = pl.pallas_call(
        _onehot_t_body,
        grid=(N_SPECIES // SUB, pl.cdiv(N_NODES, LCH)),
        in_specs=[pl.BlockSpec((1, LCH), lambda i, j: (0, j))],
        out_specs=pl.BlockSpec((SUB, LCH), lambda i, j: (i, j)),
        out_shape=jax.ShapeDtypeStruct((N_SPECIES, N_NODES), jnp.float32),
    )(idx_t)
    return out_t.T


# speedup vs baseline: 3.7669x; 3.7669x over previous
import jax
import jax.numpy as jnp
from jax.experimental import pallas as pl

N_NODES = 100000
N_SPECIES = 64
SUB = 8


def _onehot_t_body(idx_ref, out_ref):
    i = pl.program_id(0)
    row = idx_ref[...]
    sp = jax.lax.broadcasted_iota(jnp.int32, (SUB, N_NODES), 0) + SUB * i
    out_ref[...] = (row == sp).astype(jnp.float32)


def kernel(atom_types):
    idx_t = atom_types.T
    out_t = pl.pallas_call(
        _onehot_t_body,
        grid=(N_SPECIES // SUB,),
        in_specs=[pl.BlockSpec((1, N_NODES), lambda i: (0, 0))],
        out_specs=pl.BlockSpec((SUB, N_NODES), lambda i: (i, 0)),
        out_shape=jax.ShapeDtypeStruct((N_SPECIES, N_NODES), jnp.float32),
    )(idx_t)
    return out_t.T
